# baseline (speedup 1.0000x reference)
import jax
import jax.numpy as jnp
from jax.experimental import pallas as pl
from jax.experimental.pallas import tpu as pltpu

_BINS = 10
_ALPHA = 0.75
_N = 16777216
_COLS = 1024
_ROWS = _N // _COLS
_BLK_ROWS = 1024
_GRID = _ROWS // _BLK_ROWS


def _ghm_body(x_ref, t_ref, cnt_ref, ls_ref):
    i = pl.program_id(0)

    @pl.when(i == 0)
    def _init():
        for b in range(_BINS):
            cnt_ref[0, b] = 0.0
            ls_ref[0, b] = 0.0

    x = x_ref[...]
    t = t_ref[...]
    ax = jnp.abs(x)
    e = jnp.exp(-ax)
    loss = jnp.maximum(x, 0.0) - x * t + jnp.log1p(e)
    inv = 1.0 / (1.0 + e)
    pred = jnp.where(x >= 0.0, inv, e * inv)
    g = jnp.abs(pred - t)
    idx = jnp.clip(jnp.floor(g * _BINS), 0.0, _BINS - 1.0)

    for b in range(_BINS):
        m = idx == float(b)
        cnt_ref[0, b] += jnp.sum(jnp.where(m, 1.0, 0.0))
        ls_ref[0, b] += jnp.sum(jnp.where(m, loss, 0.0))


def kernel(x, target):
    x2 = x.reshape(_ROWS, _COLS)
    t2 = target.reshape(_ROWS, _COLS)
    cnt, ls = pl.pallas_call(
        _ghm_body,
        grid=(_GRID,),
        in_specs=[
            pl.BlockSpec((_BLK_ROWS, _COLS), lambda i: (i, 0)),
            pl.BlockSpec((_BLK_ROWS, _COLS), lambda i: (i, 0)),
        ],
        out_specs=[
            pl.BlockSpec(memory_space=pltpu.SMEM),
            pl.BlockSpec(memory_space=pltpu.SMEM),
        ],
        out_shape=[
            jax.ShapeDtypeStruct((1, _BINS), jnp.float32),
            jax.ShapeDtypeStruct((1, _BINS), jnp.float32),
        ],
    )(x2, t2)
    tot = jnp.clip(cnt[0], 1.0, None)
    w = tot ** (-_ALPHA)
    return jnp.sum(ls[0] * w) / _N
